# baseline (speedup 1.0000x reference)
import functools

import jax
import jax.numpy as jnp
from jax import lax
from jax.experimental import pallas as pl
from jax.experimental.pallas import tpu as pltpu
from jax.experimental.pallas import tpu_sc as plsc

N = 100000
D = 128
G = 512
TC_OUT = 24

NC = 2
NS = 16
NW = NC * NS

B = 400
NCH = 5
CH = B // NCH
NB = N // B
ROWS_PER_TILE = G // NS

HB = 4000
NHB = N // HB
HI = 32
LO = 16


def _sc_body(nodes_hbm, gi_hbm, sums_hbm,
             rows_v0, rows_v1, idx_v0, idx_v1, zrow_v, acc_sh,
             lsem0, lsem1, ssem0, ssem1):
  cid = lax.axis_index("c")
  sid = lax.axis_index("s")
  wid = cid * NS + sid

  zvec = jnp.zeros((16,), jnp.float32)

  def fill_zrow(i, _):
    for j in range(D // 16):
      zrow_v[i, pl.ds(j * 16, 16)] = zvec
    return 0
  lax.fori_loop(0, ROWS_PER_TILE, fill_zrow, 0)

  pltpu.sync_copy(zrow_v, acc_sh.at[pl.ds(sid * ROWS_PER_TILE, ROWS_PER_TILE)])
  plsc.subcore_barrier()

  nbase = NB // NW
  nrem = NB - nbase * NW
  start = nbase * wid + jnp.minimum(wid, nrem)
  nb = nbase + jnp.where(wid < nrem, 1, 0)

  def issue_load(k, rows_v, idx_v, lsem):
    bt = start + k
    pltpu.async_copy(gi_hbm.at[bt], idx_v, lsem)
    pltpu.async_copy(nodes_hbm.at[pl.ds(bt * B, B)], rows_v, lsem)

  def half(k, rows_v, idx_v, lsem, ssem):
    pltpu.make_async_copy(gi_hbm.at[0], idx_v, lsem).wait()
    pltpu.make_async_copy(nodes_hbm.at[pl.ds(0, B)], rows_v, lsem).wait()
    descs = []
    for j in range(NCH):
      descs.append(pltpu.async_copy(rows_v.at[pl.ds(j * CH, CH)],
                                    acc_sh.at[idx_v.at[j]],
                                    ssem, add=True))

    for d_ in descs:
      d_.wait()

    @pl.when(k + 2 < nb)
    def _():
      issue_load(k + 2, rows_v, idx_v, lsem)

  issue_load(0, rows_v0, idx_v0, lsem0)
  issue_load(1, rows_v1, idx_v1, lsem1)

  def pair_body(k2, _):
    k = 2 * k2

    @pl.when(k < nb)
    def _():
      half(k, rows_v0, idx_v0, lsem0, ssem0)

    @pl.when(k + 1 < nb)
    def _():
      half(k + 1, rows_v1, idx_v1, lsem1, ssem1)
    return 0

  lax.fori_loop(0, (nb + 1) // 2, pair_body, 0)
  plsc.subcore_barrier()

  out_base = cid * G + sid * ROWS_PER_TILE
  pltpu.sync_copy(acc_sh.at[pl.ds(sid * ROWS_PER_TILE, ROWS_PER_TILE)],
                  sums_hbm.at[pl.ds(out_base, ROWS_PER_TILE)])


@jax.jit
def _segment_accumulate(nodes, gi_r):
  mesh = plsc.VectorSubcoreMesh(core_axis_name="c", subcore_axis_name="s",
                                num_cores=NC, num_subcores=NS)
  return pl.kernel(
      _sc_body,
      out_type=jax.ShapeDtypeStruct((NC * G, D), jnp.float32),
      mesh=mesh,
      scratch_types=[
          pltpu.VMEM((B, D), jnp.float32),
          pltpu.VMEM((B, D), jnp.float32),
          pltpu.VMEM((NCH, CH), jnp.int32),
          pltpu.VMEM((NCH, CH), jnp.int32),
          pltpu.VMEM((ROWS_PER_TILE, D), jnp.float32),
          pltpu.VMEM_SHARED((G, D), jnp.float32),
          pltpu.SemaphoreType.DMA,
          pltpu.SemaphoreType.DMA,
          pltpu.SemaphoreType.DMA,
          pltpu.SemaphoreType.DMA,
      ],
  )(nodes, gi_r)


def _hist_body(gi_ref, cnt_ref):
  i = pl.program_id(0)
  g = gi_ref[0, 0, :]
  hi = jnp.right_shift(g, 4)
  lo = jnp.bitwise_and(g, 15)
  eqh = (hi[:, None] == lax.broadcasted_iota(jnp.int32, (HB, HI), 1)
         ).astype(jnp.bfloat16)
  eql = (lo[:, None] == lax.broadcasted_iota(jnp.int32, (HB, LO), 1)
         ).astype(jnp.bfloat16)
  bc = lax.dot_general(eqh, eql, (((0,), (0,)), ((), ())),
                       preferred_element_type=jnp.float32)

  @pl.when(i == 0)
  def _():
    cnt_ref[...] = bc

  @pl.when(i > 0)
  def _():
    cnt_ref[...] = cnt_ref[...] + bc


@jax.jit
def _hist(gi4):
  return pl.pallas_call(
      _hist_body,
      grid=(NHB,),
      in_specs=[pl.BlockSpec((1, 1, HB), lambda i: (i, 0, 0))],
      out_specs=pl.BlockSpec((HI, LO), lambda i: (0, 0)),
      out_shape=jax.ShapeDtypeStruct((HI, LO), jnp.float32),
  )(gi4)


def _head_body(sums_ref, cnt_ref, w_ref, b_ref, out_ref):
  s = sums_ref[0:G, :] + sums_ref[G:2 * G, :]
  pooled = s / jnp.maximum(cnt_ref[...], 1.0)
  out_ref[...] = (
      jnp.dot(pooled, w_ref[...], preferred_element_type=jnp.float32)
      + b_ref[...])


@jax.jit
def _head(sums, cnts, W, b):
  return pl.pallas_call(
      _head_body,
      out_shape=jax.ShapeDtypeStruct((G, TC_OUT), jnp.float32),
  )(sums, cnts, W, b.reshape(1, TC_OUT))


def kernel(node_representation, graph_index, W, b):
  gi = graph_index.astype(jnp.int32)
  gi_r = gi.reshape(NB, NCH, CH)
  gi4 = gi.reshape(NHB, 1, HB)
  sums = _segment_accumulate(node_representation, gi_r)
  cnts = _hist(gi4).reshape(G, 1)
  out = _head(sums, cnts, W, b)
  return out.reshape(-1, TC_OUT // 2, 2)


# speedup vs baseline: 1.1230x; 1.1230x over previous
import functools

import jax
import jax.numpy as jnp
from jax import lax
from jax.experimental import pallas as pl
from jax.experimental.pallas import tpu as pltpu
from jax.experimental.pallas import tpu_sc as plsc

N = 100000
D = 128
G = 512
TC_OUT = 24

NC = 2
NS = 16
NW = NC * NS

B = 200
NCH = 2
CH = B // NCH
NB = N // B
ROWS_PER_TILE = G // NS

HB = 4000
NHB = N // HB
HI = 32
LO = 16


def _sc_body(nodes_hbm, gi_hbm, sums_hbm,
             rows_v0, rows_v1, rows_v2, idx_v0, idx_v1, idx_v2, zrow_v,
             acc_sh, lsem0, lsem1, lsem2, ssem0, ssem1, ssem2):
  cid = lax.axis_index("c")
  sid = lax.axis_index("s")
  wid = cid * NS + sid

  zvec = jnp.zeros((16,), jnp.float32)

  def fill_zrow(i, _):
    for j in range(D // 16):
      zrow_v[i, pl.ds(j * 16, 16)] = zvec
    return 0
  lax.fori_loop(0, ROWS_PER_TILE, fill_zrow, 0)

  pltpu.sync_copy(zrow_v, acc_sh.at[pl.ds(sid * ROWS_PER_TILE, ROWS_PER_TILE)])
  plsc.subcore_barrier()

  nbase = NB // NW
  nrem = NB - nbase * NW
  start = nbase * wid + jnp.minimum(wid, nrem)
  nb = nbase + jnp.where(wid < nrem, 1, 0)

  slots = ((rows_v0, idx_v0, lsem0, ssem0),
           (rows_v1, idx_v1, lsem1, ssem1),
           (rows_v2, idx_v2, lsem2, ssem2))

  def issue_load(k, slot):
    rows_v, idx_v, lsem, _ = slot
    bt = start + k
    pltpu.async_copy(gi_hbm.at[bt], idx_v, lsem)
    pltpu.async_copy(nodes_hbm.at[pl.ds(bt * B, B)], rows_v, lsem)

  def fire_scatters(slot):
    rows_v, idx_v, _, ssem = slot
    for j in range(NCH):
      pltpu.async_copy(rows_v.at[pl.ds(j * CH, CH)],
                       acc_sh.at[idx_v.at[j]], ssem, add=True)

  def drain_scatters(slot):
    rows_v, idx_v, _, ssem = slot
    for j in range(NCH):
      pltpu.make_async_copy(rows_v.at[pl.ds(j * CH, CH)],
                            acc_sh.at[idx_v.at[j]], ssem).wait()

  def step(k, slot, prev_slot):
    rows_v, idx_v, lsem, _ = slot
    pltpu.make_async_copy(gi_hbm.at[0], idx_v, lsem).wait()
    pltpu.make_async_copy(nodes_hbm.at[pl.ds(0, B)], rows_v, lsem).wait()
    fire_scatters(slot)

    @pl.when(k >= 1)
    def _():
      drain_scatters(prev_slot)

    @pl.when(k + 2 < nb)
    def _():
      issue_load(k + 2, prev_slot)

  issue_load(0, slots[0])
  issue_load(1, slots[1])

  def tri_body(k3, _):
    for u in range(3):
      k = 3 * k3 + u

      @pl.when(k < nb)
      def _():
        step(k, slots[u], slots[(u + 2) % 3])
    return 0

  lax.fori_loop(0, (nb + 2) // 3, tri_body, 0)

  @pl.when(nb == nbase)
  def _():
    drain_scatters(slots[(nbase - 1) % 3])

  @pl.when(nb == nbase + 1)
  def _():
    drain_scatters(slots[nbase % 3])

  plsc.subcore_barrier()

  out_base = cid * G + sid * ROWS_PER_TILE
  pltpu.sync_copy(acc_sh.at[pl.ds(sid * ROWS_PER_TILE, ROWS_PER_TILE)],
                  sums_hbm.at[pl.ds(out_base, ROWS_PER_TILE)])


@jax.jit
def _segment_accumulate(nodes, gi_r):
  mesh = plsc.VectorSubcoreMesh(core_axis_name="c", subcore_axis_name="s",
                                num_cores=NC, num_subcores=NS)
  return pl.kernel(
      _sc_body,
      out_type=jax.ShapeDtypeStruct((NC * G, D), jnp.float32),
      mesh=mesh,
      scratch_types=[
          pltpu.VMEM((B, D), jnp.float32),
          pltpu.VMEM((B, D), jnp.float32),
          pltpu.VMEM((B, D), jnp.float32),
          pltpu.VMEM((NCH, CH), jnp.int32),
          pltpu.VMEM((NCH, CH), jnp.int32),
          pltpu.VMEM((NCH, CH), jnp.int32),
          pltpu.VMEM((ROWS_PER_TILE, D), jnp.float32),
          pltpu.VMEM_SHARED((G, D), jnp.float32),
          pltpu.SemaphoreType.DMA,
          pltpu.SemaphoreType.DMA,
          pltpu.SemaphoreType.DMA,
          pltpu.SemaphoreType.DMA,
          pltpu.SemaphoreType.DMA,
          pltpu.SemaphoreType.DMA,
      ],
  )(nodes, gi_r)


def _hist_body(gi_ref, cnt_ref):
  i = pl.program_id(0)
  g = gi_ref[0, 0, :]
  hi = jnp.right_shift(g, 4)[None, :]
  lo = jnp.bitwise_and(g, 15)[None, :]
  eqh = (hi == lax.broadcasted_iota(jnp.int32, (HI, HB), 0)
         ).astype(jnp.bfloat16)
  eql = (lo == lax.broadcasted_iota(jnp.int32, (LO, HB), 0)
         ).astype(jnp.bfloat16)
  bc = lax.dot_general(eqh, eql, (((1,), (1,)), ((), ())),
                       preferred_element_type=jnp.float32)

  @pl.when(i == 0)
  def _():
    cnt_ref[...] = bc

  @pl.when(i > 0)
  def _():
    cnt_ref[...] = cnt_ref[...] + bc


@jax.jit
def _hist(gi4):
  return pl.pallas_call(
      _hist_body,
      grid=(NHB,),
      in_specs=[pl.BlockSpec((1, 1, HB), lambda i: (i, 0, 0))],
      out_specs=pl.BlockSpec((HI, LO), lambda i: (0, 0)),
      out_shape=jax.ShapeDtypeStruct((HI, LO), jnp.float32),
  )(gi4)


def _head_body(sums_ref, cnt_ref, w_ref, b_ref, out_ref):
  s = sums_ref[0:G, :] + sums_ref[G:2 * G, :]
  pooled = s / jnp.maximum(cnt_ref[...], 1.0)
  out_ref[...] = (
      jnp.dot(pooled, w_ref[...], preferred_element_type=jnp.float32)
      + b_ref[...])


@jax.jit
def _head(sums, cnts, W, b):
  return pl.pallas_call(
      _head_body,
      out_shape=jax.ShapeDtypeStruct((G, TC_OUT), jnp.float32),
  )(sums, cnts, W, b.reshape(1, TC_OUT))


def kernel(node_representation, graph_index, W, b):
  gi = graph_index.astype(jnp.int32)
  gi_r = gi.reshape(NB, NCH, CH)
  gi4 = gi.reshape(NHB, 1, HB)
  sums = _segment_accumulate(node_representation, gi_r)
  cnts = _hist(gi4).reshape(G, 1)
  out = _head(sums, cnts, W, b)
  return out.reshape(-1, TC_OUT // 2, 2)


# speedup vs baseline: 1.1269x; 1.0035x over previous
import functools

import jax
import jax.numpy as jnp
from jax import lax
from jax.experimental import pallas as pl
from jax.experimental.pallas import tpu as pltpu
from jax.experimental.pallas import tpu_sc as plsc

N = 100000
D = 128
G = 512
TC_OUT = 24

NC = 2
NS = 16
NW = NC * NS

B = 200
CHA = 96
CHB = B - CHA
NB = N // B
ROWS_PER_TILE = G // NS

HB = 4000
NHB = N // HB
HI = 32
LO = 16


def _sc_body(nodes_hbm, gi_hbm, sums_hbm,
             rows_v0, rows_v1, rows_v2, idxa_v0, idxa_v1, idxa_v2,
             idxb_v0, idxb_v1, idxb_v2, zrow_v,
             acc_sh, lsem0, lsem1, lsem2, ssem0, ssem1, ssem2):
  cid = lax.axis_index("c")
  sid = lax.axis_index("s")
  wid = cid * NS + sid

  zvec = jnp.zeros((16,), jnp.float32)

  def fill_zrow(i, _):
    for j in range(D // 16):
      zrow_v[i, pl.ds(j * 16, 16)] = zvec
    return 0
  lax.fori_loop(0, ROWS_PER_TILE, fill_zrow, 0)

  pltpu.sync_copy(zrow_v, acc_sh.at[pl.ds(sid * ROWS_PER_TILE, ROWS_PER_TILE)])
  plsc.subcore_barrier()

  nbase = NB // NW
  nrem = NB - nbase * NW
  start = nbase * wid + jnp.minimum(wid, nrem)
  nb = nbase + jnp.where(wid < nrem, 1, 0)

  slots = ((rows_v0, idxa_v0, idxb_v0, lsem0, ssem0),
           (rows_v1, idxa_v1, idxb_v1, lsem1, ssem1),
           (rows_v2, idxa_v2, idxb_v2, lsem2, ssem2))

  def issue_load(k, slot):
    rows_v, idxa_v, idxb_v, lsem, _ = slot
    bt = start + k
    pltpu.async_copy(gi_hbm.at[pl.ds(bt * B, CHA)], idxa_v, lsem)
    pltpu.async_copy(gi_hbm.at[pl.ds(bt * B + CHA, CHB)], idxb_v, lsem)
    pltpu.async_copy(nodes_hbm.at[pl.ds(bt * B, B)], rows_v, lsem)

  def fire_scatters(slot):
    rows_v, idxa_v, idxb_v, _, ssem = slot
    pltpu.async_copy(rows_v.at[pl.ds(0, CHA)], acc_sh.at[idxa_v],
                     ssem, add=True)
    pltpu.async_copy(rows_v.at[pl.ds(CHA, CHB)], acc_sh.at[idxb_v],
                     ssem, add=True)

  def drain_scatters(slot):
    rows_v, idxa_v, idxb_v, _, ssem = slot
    pltpu.make_async_copy(rows_v.at[pl.ds(0, CHA)], acc_sh.at[idxa_v],
                          ssem).wait()
    pltpu.make_async_copy(rows_v.at[pl.ds(CHA, CHB)], acc_sh.at[idxb_v],
                          ssem).wait()

  def step(k, slot, prev_slot):
    rows_v, idxa_v, idxb_v, lsem, _ = slot
    pltpu.make_async_copy(gi_hbm.at[pl.ds(0, CHA)], idxa_v, lsem).wait()
    pltpu.make_async_copy(gi_hbm.at[pl.ds(0, CHB)], idxb_v, lsem).wait()
    pltpu.make_async_copy(nodes_hbm.at[pl.ds(0, B)], rows_v, lsem).wait()
    fire_scatters(slot)

    @pl.when(k >= 1)
    def _():
      drain_scatters(prev_slot)

    @pl.when(k + 2 < nb)
    def _():
      issue_load(k + 2, prev_slot)

  issue_load(0, slots[0])
  issue_load(1, slots[1])

  def tri_body(k3, _):
    for u in range(3):
      k = 3 * k3 + u

      @pl.when(k < nb)
      def _():
        step(k, slots[u], slots[(u + 2) % 3])
    return 0

  lax.fori_loop(0, (nb + 2) // 3, tri_body, 0)

  @pl.when(nb == nbase)
  def _():
    drain_scatters(slots[(nbase - 1) % 3])

  @pl.when(nb == nbase + 1)
  def _():
    drain_scatters(slots[nbase % 3])

  plsc.subcore_barrier()

  out_base = cid * G + sid * ROWS_PER_TILE
  pltpu.sync_copy(acc_sh.at[pl.ds(sid * ROWS_PER_TILE, ROWS_PER_TILE)],
                  sums_hbm.at[pl.ds(out_base, ROWS_PER_TILE)])


@jax.jit
def _segment_accumulate(nodes, gi_r):
  mesh = plsc.VectorSubcoreMesh(core_axis_name="c", subcore_axis_name="s",
                                num_cores=NC, num_subcores=NS)
  return pl.kernel(
      _sc_body,
      out_type=jax.ShapeDtypeStruct((NC * G, D), jnp.float32),
      mesh=mesh,
      scratch_types=[
          pltpu.VMEM((B, D), jnp.float32),
          pltpu.VMEM((B, D), jnp.float32),
          pltpu.VMEM((B, D), jnp.float32),
          pltpu.VMEM((CHA,), jnp.int32),
          pltpu.VMEM((CHA,), jnp.int32),
          pltpu.VMEM((CHA,), jnp.int32),
          pltpu.VMEM((CHB,), jnp.int32),
          pltpu.VMEM((CHB,), jnp.int32),
          pltpu.VMEM((CHB,), jnp.int32),
          pltpu.VMEM((ROWS_PER_TILE, D), jnp.float32),
          pltpu.VMEM_SHARED((G, D), jnp.float32),
          pltpu.SemaphoreType.DMA,
          pltpu.SemaphoreType.DMA,
          pltpu.SemaphoreType.DMA,
          pltpu.SemaphoreType.DMA,
          pltpu.SemaphoreType.DMA,
          pltpu.SemaphoreType.DMA,
      ],
  )(nodes, gi_r)


def _hist_body(gi_ref, cnt_ref):
  i = pl.program_id(0)
  g = gi_ref[0, 0, :]
  hi = jnp.right_shift(g, 4)[None, :]
  lo = jnp.bitwise_and(g, 15)[None, :]
  eqh = (hi == lax.broadcasted_iota(jnp.int32, (HI, HB), 0)
         ).astype(jnp.bfloat16)
  eql = (lo == lax.broadcasted_iota(jnp.int32, (LO, HB), 0)
         ).astype(jnp.bfloat16)
  bc = lax.dot_general(eqh, eql, (((1,), (1,)), ((), ())),
                       preferred_element_type=jnp.float32)

  @pl.when(i == 0)
  def _():
    cnt_ref[...] = bc

  @pl.when(i > 0)
  def _():
    cnt_ref[...] = cnt_ref[...] + bc


@jax.jit
def _hist(gi4):
  return pl.pallas_call(
      _hist_body,
      grid=(NHB,),
      in_specs=[pl.BlockSpec((1, 1, HB), lambda i: (i, 0, 0))],
      out_specs=pl.BlockSpec((HI, LO), lambda i: (0, 0)),
      out_shape=jax.ShapeDtypeStruct((HI, LO), jnp.float32),
  )(gi4)


def _head_body(sums_ref, cnt_ref, w_ref, b_ref, out_ref):
  s = sums_ref[0:G, :] + sums_ref[G:2 * G, :]
  pooled = s / jnp.maximum(cnt_ref[...], 1.0)
  out_ref[...] = (
      jnp.dot(pooled, w_ref[...], preferred_element_type=jnp.float32)
      + b_ref[...])


@jax.jit
def _head(sums, cnts, W, b):
  return pl.pallas_call(
      _head_body,
      out_shape=jax.ShapeDtypeStruct((G, TC_OUT), jnp.float32),
  )(sums, cnts, W, b.reshape(1, TC_OUT))


def kernel(node_representation, graph_index, W, b):
  gi = graph_index.astype(jnp.int32)
  gi4 = gi.reshape(NHB, 1, HB)
  sums = _segment_accumulate(node_representation, gi)
  cnts = _hist(gi4).reshape(G, 1)
  out = _head(sums, cnts, W, b)
  return out.reshape(-1, TC_OUT // 2, 2)


# speedup vs baseline: 1.1560x; 1.0258x over previous
import functools

import jax
import jax.numpy as jnp
from jax import lax
from jax.experimental import pallas as pl
from jax.experimental.pallas import tpu as pltpu
from jax.experimental.pallas import tpu_sc as plsc

N = 100000
D = 128
G = 512
TC_OUT = 24

NC = 2
NS = 16
NW = NC * NS

B = 400
CHUNKS = ((0, 96), (96, 104), (200, 96), (296, 104))
NB = N // B
ROWS_PER_TILE = G // NS

HB = 4000
NHB = N // HB
HI = 32
LO = 16


def _sc_body(nodes_hbm, gi_hbm, sums_hbm,
             rows_v0, rows_v1,
             idxa_v0, idxb_v0, idxc_v0, idxd_v0,
             idxa_v1, idxb_v1, idxc_v1, idxd_v1, zrow_v,
             acc_sh, lsem0, lsem1, ssem0, ssem1):
  cid = lax.axis_index("c")
  sid = lax.axis_index("s")
  wid = cid * NS + sid

  zvec = jnp.zeros((16,), jnp.float32)

  def fill_zrow(i, _):
    for j in range(D // 16):
      zrow_v[i, pl.ds(j * 16, 16)] = zvec
    return 0
  lax.fori_loop(0, ROWS_PER_TILE, fill_zrow, 0)

  pltpu.sync_copy(zrow_v, acc_sh.at[pl.ds(sid * ROWS_PER_TILE, ROWS_PER_TILE)])
  plsc.subcore_barrier()

  nbase = NB // NW
  nrem = NB - nbase * NW
  start = nbase * wid + jnp.minimum(wid, nrem)
  nb = nbase + jnp.where(wid < nrem, 1, 0)

  slots = ((rows_v0, (idxa_v0, idxb_v0, idxc_v0, idxd_v0), lsem0, ssem0),
           (rows_v1, (idxa_v1, idxb_v1, idxc_v1, idxd_v1), lsem1, ssem1))

  def issue_load(k, slot):
    rows_v, idxs, lsem, _ = slot
    bt = start + k
    for (off, ln), iv in zip(CHUNKS, idxs):
      pltpu.async_copy(gi_hbm.at[pl.ds(bt * B + off, ln)], iv, lsem)
    pltpu.async_copy(nodes_hbm.at[pl.ds(bt * B, B)], rows_v, lsem)

  def half(k, slot):
    rows_v, idxs, lsem, ssem = slot
    for (off, ln), iv in zip(CHUNKS, idxs):
      pltpu.make_async_copy(gi_hbm.at[pl.ds(0, ln)], iv, lsem).wait()
    pltpu.make_async_copy(nodes_hbm.at[pl.ds(0, B)], rows_v, lsem).wait()
    descs = []
    for (off, ln), iv in zip(CHUNKS, idxs):
      descs.append(pltpu.async_copy(rows_v.at[pl.ds(off, ln)],
                                    acc_sh.at[iv], ssem, add=True))
    for d_ in descs:
      d_.wait()

    @pl.when(k + 2 < nb)
    def _():
      issue_load(k + 2, slot)

  issue_load(0, slots[0])
  issue_load(1, slots[1])

  def pair_body(k2, _):
    k = 2 * k2

    @pl.when(k < nb)
    def _():
      half(k, slots[0])

    @pl.when(k + 1 < nb)
    def _():
      half(k + 1, slots[1])
    return 0

  lax.fori_loop(0, (nb + 1) // 2, pair_body, 0)
  plsc.subcore_barrier()

  out_base = cid * G + sid * ROWS_PER_TILE
  pltpu.sync_copy(acc_sh.at[pl.ds(sid * ROWS_PER_TILE, ROWS_PER_TILE)],
                  sums_hbm.at[pl.ds(out_base, ROWS_PER_TILE)])


@jax.jit
def _segment_accumulate(nodes, gi_r):
  mesh = plsc.VectorSubcoreMesh(core_axis_name="c", subcore_axis_name="s",
                                num_cores=NC, num_subcores=NS)
  return pl.kernel(
      _sc_body,
      out_type=jax.ShapeDtypeStruct((NC * G, D), jnp.float32),
      mesh=mesh,
      scratch_types=[
          pltpu.VMEM((B, D), jnp.float32),
          pltpu.VMEM((B, D), jnp.float32),
          pltpu.VMEM((96,), jnp.int32),
          pltpu.VMEM((104,), jnp.int32),
          pltpu.VMEM((96,), jnp.int32),
          pltpu.VMEM((104,), jnp.int32),
          pltpu.VMEM((96,), jnp.int32),
          pltpu.VMEM((104,), jnp.int32),
          pltpu.VMEM((96,), jnp.int32),
          pltpu.VMEM((104,), jnp.int32),
          pltpu.VMEM((ROWS_PER_TILE, D), jnp.float32),
          pltpu.VMEM_SHARED((G, D), jnp.float32),
          pltpu.SemaphoreType.DMA,
          pltpu.SemaphoreType.DMA,
          pltpu.SemaphoreType.DMA,
          pltpu.SemaphoreType.DMA,
      ],
  )(nodes, gi_r)


def _hist_body(gi_ref, cnt_ref):
  i = pl.program_id(0)
  g = gi_ref[0, 0, :]
  hi = jnp.right_shift(g, 4)[None, :]
  lo = jnp.bitwise_and(g, 15)[None, :]
  eqh = (hi == lax.broadcasted_iota(jnp.int32, (HI, HB), 0)
         ).astype(jnp.bfloat16)
  eql = (lo == lax.broadcasted_iota(jnp.int32, (LO, HB), 0)
         ).astype(jnp.bfloat16)
  bc = lax.dot_general(eqh, eql, (((1,), (1,)), ((), ())),
                       preferred_element_type=jnp.float32)

  @pl.when(i == 0)
  def _():
    cnt_ref[...] = bc

  @pl.when(i > 0)
  def _():
    cnt_ref[...] = cnt_ref[...] + bc


@jax.jit
def _hist(gi4):
  return pl.pallas_call(
      _hist_body,
      grid=(NHB,),
      in_specs=[pl.BlockSpec((1, 1, HB), lambda i: (i, 0, 0))],
      out_specs=pl.BlockSpec((HI, LO), lambda i: (0, 0)),
      out_shape=jax.ShapeDtypeStruct((HI, LO), jnp.float32),
  )(gi4)


def _head_body(sums_ref, cnt_ref, w_ref, b_ref, out_ref):
  s = sums_ref[0:G, :] + sums_ref[G:2 * G, :]
  pooled = s / jnp.maximum(cnt_ref[...], 1.0)
  out_ref[...] = (
      jnp.dot(pooled, w_ref[...], preferred_element_type=jnp.float32)
      + b_ref[...])


@jax.jit
def _head(sums, cnts, W, b):
  return pl.pallas_call(
      _head_body,
      out_shape=jax.ShapeDtypeStruct((G, TC_OUT), jnp.float32),
  )(sums, cnts, W, b.reshape(1, TC_OUT))


def kernel(node_representation, graph_index, W, b):
  gi = graph_index.astype(jnp.int32)
  gi4 = gi.reshape(NHB, 1, HB)
  sums = _segment_accumulate(node_representation, gi)
  cnts = _hist(gi4).reshape(G, 1)
  out = _head(sums, cnts, W, b)
  return out.reshape(-1, TC_OUT // 2, 2)
